# baseline (speedup 1.0000x reference)
import functools

import jax
import jax.numpy as jnp
from jax import lax
from jax.experimental import pallas as pl
from jax.experimental.pallas import tpu as pltpu
from jax.experimental.pallas import tpu_sc as plsc

N = 10000
E = 160000
D = 256
H = 256
B = 64
HALF = 128
NT = 16
EC = 125
NP_PAD = 10240
RPT = NP_PAD // NT
CB = 128
RB = RPT // CB
MSG_CHUNKS = E // NT // EC
SLAB = 40
DEG_CHUNKS = E // (2 * NT) // EC
SEG = NP_PAD // NT
BM = 1000
G = N // BM
EPS = 1e-5

_mesh = plsc.VectorSubcoreMesh(core_axis_name="c", subcore_axis_name="s")


def _deg_body(dst3, hist_out, idx_vm, buf, hist):
    c = lax.axis_index("c")
    s = lax.axis_index("s")
    w = c * NT + s
    pltpu.sync_copy(dst3.at[w], idx_vm)

    def _zero(j, carry):
        buf[pl.ds(j * 16, 16)] = jnp.zeros((16,), jnp.float32)
        return carry
    lax.fori_loop(0, SEG // 16, _zero, None)
    seg = s * SEG
    pltpu.sync_copy(buf, hist.at[pl.ds(seg, SEG)])
    plsc.subcore_barrier()

    def _ones(j, carry):
        buf[pl.ds(j * 16, 16)] = jnp.ones((16,), jnp.float32)
        return carry
    lax.fori_loop(0, 8, _ones, None)

    def _chunk(j, carry):
        pltpu.sync_copy(buf.at[pl.ds(0, EC)], hist.at[idx_vm.at[j]], add=True)
        return carry
    lax.fori_loop(0, DEG_CHUNKS, _chunk, None)
    plsc.subcore_barrier()
    pltpu.sync_copy(hist.at[pl.ds(seg, SEG)], buf)
    pltpu.sync_copy(buf, hist_out.at[c, pl.ds(seg, SEG)])


_deg_call = pl.kernel(
    _deg_body,
    out_type=jax.ShapeDtypeStruct((2, NP_PAD), jnp.float32),
    mesh=_mesh,
    scratch_types=[
        pltpu.VMEM((DEG_CHUNKS, EC), jnp.int32),
        pltpu.VMEM((SEG,), jnp.float32),
        pltpu.VMEM_SHARED((NP_PAD,), jnp.float32),
    ],
)


def _msg_body(y_lo, y_hi, src3, dst3, s_lo, s_hi, src_vm, dst_vm, rows0, rows1,
              acc, sem0, sem1, ssem0, ssem1):
    c = lax.axis_index("c")
    s = lax.axis_index("s")

    def _zrow(j, carry):
        for k in range(HALF // 16):
            rows0[j, pl.ds(k * 16, 16)] = jnp.zeros((16,), jnp.float32)
        return carry
    lax.fori_loop(0, CB, _zrow, None)
    base = s * RPT
    for k in range(RB):
        pltpu.sync_copy(rows0, acc.at[pl.ds(base + k * CB, CB)])
    plsc.subcore_barrier()

    bufs = (rows0.at[pl.ds(0, EC)], rows1.at[pl.ds(0, EC)])
    sems = (sem0, sem1)
    ssems = (ssem0, ssem1)

    def _pass(tbl, out):
        def _gather(ch, b):
            pltpu.async_copy(tbl.at[src_vm.at[ch]], bufs[b], sems[b])

        def _gather_wait(ch, b):
            pltpu.make_async_copy(tbl.at[src_vm.at[ch]], bufs[b], sems[b]).wait()

        def _scat(ch, b):
            pltpu.async_copy(bufs[b], acc.at[dst_vm.at[ch]], ssems[b], add=True)

        def _scat_wait(ch, b):
            pltpu.make_async_copy(bufs[b], acc.at[dst_vm.at[ch]], ssems[b]).wait()

        def _step(ch, b):
            _gather_wait(ch, b)
            _scat(ch, b)
            _scat_wait(ch - 1, 1 - b)
            _gather(ch + 1, 1 - b)

        for h in range(MSG_CHUNKS // SLAB):
            pltpu.sync_copy(src3.at[s, pl.ds(h * SLAB, SLAB)], src_vm)
            pltpu.sync_copy(dst3.at[s, pl.ds(h * SLAB, SLAB)], dst_vm)
            _gather(0, 0)
            _gather_wait(0, 0)
            _scat(0, 0)
            _gather(1, 1)

            def _loop(j, carry):
                _step(2 * j + 1, 1)
                _step(2 * j + 2, 0)
                return carry
            lax.fori_loop(0, SLAB // 2 - 1, _loop, None)
            _gather_wait(SLAB - 1, 1)
            _scat(SLAB - 1, 1)
            _scat_wait(SLAB - 2, 0)
            _scat_wait(SLAB - 1, 1)

        plsc.subcore_barrier()
        pltpu.sync_copy(acc.at[pl.ds(base, RPT)], out.at[pl.ds(base, RPT)])

    pl.when(c == 0)(lambda: _pass(y_lo, s_lo))
    pl.when(c == 1)(lambda: _pass(y_hi, s_hi))


_msg_call = pl.kernel(
    _msg_body,
    out_type=[jax.ShapeDtypeStruct((NP_PAD, HALF), jnp.float32)] * 2,
    mesh=_mesh,
    scratch_types=[
        pltpu.VMEM((SLAB, EC), jnp.int32),
        pltpu.VMEM((SLAB, EC), jnp.int32),
        pltpu.VMEM((CB, HALF), jnp.float32),
        pltpu.VMEM((EC, HALF), jnp.float32),
        pltpu.VMEM_SHARED((NP_PAD, HALF), jnp.float32),
        pltpu.SemaphoreType.DMA,
        pltpu.SemaphoreType.DMA,
        pltpu.SemaphoreType.DMA,
        pltpu.SemaphoreType.DMA,
    ],
)


def _dinv_body(h_ref, o_ref):
    deg = h_ref[0:1, :] + h_ref[1:2, :] + 1.0
    o_ref[...] = lax.rsqrt(deg)


_dinv_call = pl.pallas_call(
    _dinv_body,
    out_shape=jax.ShapeDtypeStruct((1, NP_PAD), jnp.float32),
)


def _mm1_body(x_ref, w_ref, dinv_ref, ylo_ref, yhi_ref):
    t = jnp.dot(x_ref[...], w_ref[...], preferred_element_type=jnp.float32)
    y = t * dinv_ref[...]
    ylo_ref[...] = y[:, :HALF]
    yhi_ref[...] = y[:, HALF:]


_mm1_call = pl.pallas_call(
    _mm1_body,
    grid=(G,),
    in_specs=[
        pl.BlockSpec((BM, D), lambda i: (i, 0)),
        pl.BlockSpec((D, H), lambda i: (0, 0)),
        pl.BlockSpec((BM, 1), lambda i: (i, 0)),
    ],
    out_specs=[pl.BlockSpec((BM, HALF), lambda i: (i, 0))] * 2,
    out_shape=[jax.ShapeDtypeStruct((N, HALF), jnp.float32)] * 2,
)


def _mm_norm_body(z_ref, st_ref, g_ref, bt_ref, w_ref, dinv_ref, ylo_ref, yhi_ref):
    mean = st_ref[0:1, :] * (1.0 / N)
    var = st_ref[1:2, :] * (1.0 / N) - mean * mean
    a = g_ref[...] * lax.rsqrt(var + EPS)
    cc = bt_ref[...] - mean * a
    h = jnp.maximum(z_ref[...] * a + cc, 0.0)
    t = jnp.dot(h, w_ref[...], preferred_element_type=jnp.float32)
    y = t * dinv_ref[...]
    ylo_ref[...] = y[:, :HALF]
    yhi_ref[...] = y[:, HALF:]


_mm_norm_call = pl.pallas_call(
    _mm_norm_body,
    grid=(G,),
    in_specs=[
        pl.BlockSpec((BM, H), lambda i: (i, 0)),
        pl.BlockSpec((2, H), lambda i: (0, 0)),
        pl.BlockSpec((1, H), lambda i: (0, 0)),
        pl.BlockSpec((1, H), lambda i: (0, 0)),
        pl.BlockSpec((H, H), lambda i: (0, 0)),
        pl.BlockSpec((BM, 1), lambda i: (i, 0)),
    ],
    out_specs=[pl.BlockSpec((BM, HALF), lambda i: (i, 0))] * 2,
    out_shape=[jax.ShapeDtypeStruct((N, HALF), jnp.float32)] * 2,
)


def _combine_body(ylo_ref, yhi_ref, slo_ref, shi_ref, dinv_ref, b_ref, z_ref, st_ref):
    i = pl.program_id(0)
    d = dinv_ref[...]
    zlo = d * (slo_ref[...] + ylo_ref[...]) + b_ref[:, :HALF]
    zhi = d * (shi_ref[...] + yhi_ref[...]) + b_ref[:, HALF:]
    z = jnp.concatenate([zlo, zhi], axis=1)
    z_ref[...] = z

    @pl.when(i == 0)
    def _():
        st_ref[...] = jnp.zeros((2, H), jnp.float32)

    st_ref[...] += jnp.concatenate(
        [jnp.sum(z, 0, keepdims=True), jnp.sum(z * z, 0, keepdims=True)], axis=0)


_combine_call = pl.pallas_call(
    _combine_body,
    grid=(G,),
    in_specs=[
        pl.BlockSpec((BM, HALF), lambda i: (i, 0)),
        pl.BlockSpec((BM, HALF), lambda i: (i, 0)),
        pl.BlockSpec((BM, HALF), lambda i: (i, 0)),
        pl.BlockSpec((BM, HALF), lambda i: (i, 0)),
        pl.BlockSpec((BM, 1), lambda i: (i, 0)),
        pl.BlockSpec((1, H), lambda i: (0, 0)),
    ],
    out_specs=[
        pl.BlockSpec((BM, H), lambda i: (i, 0)),
        pl.BlockSpec((2, H), lambda i: (0, 0)),
    ],
    out_shape=[
        jax.ShapeDtypeStruct((N, H), jnp.float32),
        jax.ShapeDtypeStruct((2, H), jnp.float32),
    ],
)


def _pool_body(z_ref, st_ref, g_ref, bt_ref, batch_ref, fc1w_ref, fc1b_ref,
               fc2w_ref, fc2b_ref, o_ref, pooled, cnt):
    i = pl.program_id(0)
    mean = st_ref[0:1, :] * (1.0 / N)
    var = st_ref[1:2, :] * (1.0 / N) - mean * mean
    a = g_ref[...] * lax.rsqrt(var + EPS)
    cc = bt_ref[...] - mean * a
    h = jnp.maximum(z_ref[...] * a + cc, 0.0)
    onehot = (batch_ref[...] == lax.broadcasted_iota(jnp.int32, (BM, B), 1)
              ).astype(jnp.float32)

    @pl.when(i == 0)
    def _():
        pooled[...] = jnp.zeros((B, H), jnp.float32)
        cnt[...] = jnp.zeros((B, 1), jnp.float32)

    pooled[...] += lax.dot_general(
        onehot, h, (((0,), (0,)), ((), ())), preferred_element_type=jnp.float32)
    cnt[...] += lax.dot_general(
        onehot, jnp.ones((BM, 1), jnp.float32), (((0,), (0,)), ((), ())),
        preferred_element_type=jnp.float32)

    pm = pooled[...] / jnp.maximum(cnt[...], 1.0)
    h1 = jnp.maximum(
        jnp.dot(pm, fc1w_ref[...], preferred_element_type=jnp.float32)
        + fc1b_ref[...], 0.0)
    o_ref[...] = jnp.dot(h1, fc2w_ref[...],
                         preferred_element_type=jnp.float32) + fc2b_ref[...]


_pool_call = pl.pallas_call(
    _pool_body,
    grid=(G,),
    in_specs=[
        pl.BlockSpec((BM, H), lambda i: (i, 0)),
        pl.BlockSpec((2, H), lambda i: (0, 0)),
        pl.BlockSpec((1, H), lambda i: (0, 0)),
        pl.BlockSpec((1, H), lambda i: (0, 0)),
        pl.BlockSpec((BM, 1), lambda i: (i, 0)),
        pl.BlockSpec((H, H // 2), lambda i: (0, 0)),
        pl.BlockSpec((1, H // 2), lambda i: (0, 0)),
        pl.BlockSpec((H // 2, 1), lambda i: (0, 0)),
        pl.BlockSpec((1, 1), lambda i: (0, 0)),
    ],
    out_specs=pl.BlockSpec((B, 1), lambda i: (0, 0)),
    out_shape=jax.ShapeDtypeStruct((B, 1), jnp.float32),
    scratch_shapes=[
        pltpu.VMEM((B, H), jnp.float32),
        pltpu.VMEM((B, 1), jnp.float32),
    ],
)


def kernel(x, edge_index, batch, W0, b0, gamma0, beta0, W1, b1, gamma1, beta1,
           W2, b2, gamma2, beta2, fc1_W, fc1_b, fc2_W, fc2_b):
    src = edge_index[0].astype(jnp.int32)
    dst = edge_index[1].astype(jnp.int32)
    src3 = src.reshape(NT, MSG_CHUNKS, EC)
    dst3 = dst.reshape(NT, MSG_CHUNKS, EC)
    dstd = dst.reshape(2 * NT, DEG_CHUNKS, EC)

    hist = _deg_call(dstd)
    dinv = _dinv_call(hist).reshape(NP_PAD, 1)[:N]

    Ws = [W0, W1, W2]
    bs = [b0, b1, b2]
    gs = [gamma0, gamma1, gamma2]
    bts = [beta0, beta1, beta2]

    z = None
    st = None
    for i in range(3):
        if i == 0:
            ylo, yhi = _mm1_call(x, Ws[0], dinv)
        else:
            ylo, yhi = _mm_norm_call(z, st, gs[i - 1].reshape(1, H),
                                     bts[i - 1].reshape(1, H), Ws[i], dinv)
        slo, shi = _msg_call(ylo, yhi, src3, dst3)
        z, st = _combine_call(ylo, yhi, slo, shi, dinv, bs[i].reshape(1, H))

    out = _pool_call(z, st, gs[2].reshape(1, H), bts[2].reshape(1, H),
                     batch.astype(jnp.int32).reshape(N, 1), fc1_W,
                     fc1_b.reshape(1, H // 2), fc2_W, fc2_b.reshape(1, 1))
    return out.reshape(B)


# speedup vs baseline: 1.1345x; 1.1345x over previous
import functools

import jax
import jax.numpy as jnp
from jax import lax
from jax.experimental import pallas as pl
from jax.experimental.pallas import tpu as pltpu
from jax.experimental.pallas import tpu_sc as plsc

N = 10000
E = 160000
D = 256
H = 256
B = 64
HALF = 128
NT = 16
EC = 125
NP_PAD = 10240
RPT = NP_PAD // NT
CB = 128
RB = RPT // CB
MSG_CHUNKS = E // NT // EC
SLAB = 40
DEG_CHUNKS = E // (2 * NT) // EC
SEG = NP_PAD // NT
BM = 1000
G = N // BM
EPS = 1e-5

_mesh = plsc.VectorSubcoreMesh(core_axis_name="c", subcore_axis_name="s")


def _deg_body(dst3, hist_out, idx_vm, buf, hist):
    c = lax.axis_index("c")
    s = lax.axis_index("s")
    w = c * NT + s
    pltpu.sync_copy(dst3.at[w], idx_vm)

    def _zero(j, carry):
        buf[pl.ds(j * 16, 16)] = jnp.zeros((16,), jnp.float32)
        return carry
    lax.fori_loop(0, SEG // 16, _zero, None)
    seg = s * SEG
    pltpu.sync_copy(buf, hist.at[pl.ds(seg, SEG)])
    plsc.subcore_barrier()

    def _ones(j, carry):
        buf[pl.ds(j * 16, 16)] = jnp.ones((16,), jnp.float32)
        return carry
    lax.fori_loop(0, 8, _ones, None)

    def _chunk(j, carry):
        pltpu.sync_copy(buf.at[pl.ds(0, EC)], hist.at[idx_vm.at[j]], add=True)
        return carry
    lax.fori_loop(0, DEG_CHUNKS, _chunk, None)
    plsc.subcore_barrier()
    pltpu.sync_copy(hist.at[pl.ds(seg, SEG)], buf)
    pltpu.sync_copy(buf, hist_out.at[c, pl.ds(seg, SEG)])


_deg_call = pl.kernel(
    _deg_body,
    out_type=jax.ShapeDtypeStruct((2, NP_PAD), jnp.float32),
    mesh=_mesh,
    scratch_types=[
        pltpu.VMEM((DEG_CHUNKS, EC), jnp.int32),
        pltpu.VMEM((SEG,), jnp.float32),
        pltpu.VMEM_SHARED((NP_PAD,), jnp.float32),
    ],
)


def _msg_body(y_lo, y_hi, src3, dst3, s_lo, s_hi, src_vm, dst_vm, rows0, rows1,
              acc, sem0, sem1):
    c = lax.axis_index("c")
    s = lax.axis_index("s")

    def _zrow(j, carry):
        for k in range(HALF // 16):
            rows0[j, pl.ds(k * 16, 16)] = jnp.zeros((16,), jnp.float32)
        return carry
    lax.fori_loop(0, CB, _zrow, None)
    base = s * RPT
    for k in range(RB):
        pltpu.sync_copy(rows0, acc.at[pl.ds(base + k * CB, CB)])
    plsc.subcore_barrier()

    bufs = (rows0.at[pl.ds(0, EC)], rows1.at[pl.ds(0, EC)])
    sems = (sem0, sem1)

    def _pass(tbl, out):
        def _gather(ch, b):
            pltpu.async_copy(tbl.at[src_vm.at[ch]], bufs[b], sems[b])

        def _step(ch, b, issue_next):
            pltpu.make_async_copy(tbl.at[src_vm.at[ch]], bufs[b], sems[b]).wait()
            pltpu.sync_copy(bufs[b], acc.at[dst_vm.at[ch]], add=True)
            if issue_next:
                _gather(ch + 2, b)

        for h in range(MSG_CHUNKS // SLAB):
            pltpu.sync_copy(src3.at[s, pl.ds(h * SLAB, SLAB)], src_vm)
            pltpu.sync_copy(dst3.at[s, pl.ds(h * SLAB, SLAB)], dst_vm)
            _gather(0, 0)
            _gather(1, 1)

            def _loop(j, carry):
                _step(2 * j, 0, True)
                _step(2 * j + 1, 1, True)
                return carry
            lax.fori_loop(0, SLAB // 2 - 1, _loop, None)
            _step(SLAB - 2, 0, False)
            _step(SLAB - 1, 1, False)

        plsc.subcore_barrier()
        pltpu.sync_copy(acc.at[pl.ds(base, RPT)], out.at[pl.ds(base, RPT)])

    pl.when(c == 0)(lambda: _pass(y_lo, s_lo))
    pl.when(c == 1)(lambda: _pass(y_hi, s_hi))


_msg_call = pl.kernel(
    _msg_body,
    out_type=[jax.ShapeDtypeStruct((NP_PAD, HALF), jnp.float32)] * 2,
    mesh=_mesh,
    scratch_types=[
        pltpu.VMEM((SLAB, EC), jnp.int32),
        pltpu.VMEM((SLAB, EC), jnp.int32),
        pltpu.VMEM((CB, HALF), jnp.float32),
        pltpu.VMEM((EC, HALF), jnp.float32),
        pltpu.VMEM_SHARED((NP_PAD, HALF), jnp.float32),
        pltpu.SemaphoreType.DMA,
        pltpu.SemaphoreType.DMA,
    ],
)


def _dinv_body(h_ref, o_ref):
    deg = h_ref[0:1, :] + h_ref[1:2, :] + 1.0
    o_ref[...] = lax.rsqrt(deg)


_dinv_call = pl.pallas_call(
    _dinv_body,
    out_shape=jax.ShapeDtypeStruct((1, NP_PAD), jnp.float32),
)


def _mm1_body(x_ref, w_ref, dinv_ref, ylo_ref, yhi_ref):
    t = jnp.dot(x_ref[...], w_ref[...], preferred_element_type=jnp.float32)
    y = t * dinv_ref[...]
    ylo_ref[...] = y[:, :HALF]
    yhi_ref[...] = y[:, HALF:]


_mm1_call = pl.pallas_call(
    _mm1_body,
    grid=(G,),
    in_specs=[
        pl.BlockSpec((BM, D), lambda i: (i, 0)),
        pl.BlockSpec((D, H), lambda i: (0, 0)),
        pl.BlockSpec((BM, 1), lambda i: (i, 0)),
    ],
    out_specs=[pl.BlockSpec((BM, HALF), lambda i: (i, 0))] * 2,
    out_shape=[jax.ShapeDtypeStruct((N, HALF), jnp.float32)] * 2,
)


def _mm_norm_body(z_ref, st_ref, g_ref, bt_ref, w_ref, dinv_ref, ylo_ref, yhi_ref):
    mean = st_ref[0:1, :] * (1.0 / N)
    var = st_ref[1:2, :] * (1.0 / N) - mean * mean
    a = g_ref[...] * lax.rsqrt(var + EPS)
    cc = bt_ref[...] - mean * a
    h = jnp.maximum(z_ref[...] * a + cc, 0.0)
    t = jnp.dot(h, w_ref[...], preferred_element_type=jnp.float32)
    y = t * dinv_ref[...]
    ylo_ref[...] = y[:, :HALF]
    yhi_ref[...] = y[:, HALF:]


_mm_norm_call = pl.pallas_call(
    _mm_norm_body,
    grid=(G,),
    in_specs=[
        pl.BlockSpec((BM, H), lambda i: (i, 0)),
        pl.BlockSpec((2, H), lambda i: (0, 0)),
        pl.BlockSpec((1, H), lambda i: (0, 0)),
        pl.BlockSpec((1, H), lambda i: (0, 0)),
        pl.BlockSpec((H, H), lambda i: (0, 0)),
        pl.BlockSpec((BM, 1), lambda i: (i, 0)),
    ],
    out_specs=[pl.BlockSpec((BM, HALF), lambda i: (i, 0))] * 2,
    out_shape=[jax.ShapeDtypeStruct((N, HALF), jnp.float32)] * 2,
)


def _combine_body(ylo_ref, yhi_ref, slo_ref, shi_ref, dinv_ref, b_ref, z_ref, st_ref):
    i = pl.program_id(0)
    d = dinv_ref[...]
    zlo = d * (slo_ref[...] + ylo_ref[...]) + b_ref[:, :HALF]
    zhi = d * (shi_ref[...] + yhi_ref[...]) + b_ref[:, HALF:]
    z = jnp.concatenate([zlo, zhi], axis=1)
    z_ref[...] = z

    @pl.when(i == 0)
    def _():
        st_ref[...] = jnp.zeros((2, H), jnp.float32)

    st_ref[...] += jnp.concatenate(
        [jnp.sum(z, 0, keepdims=True), jnp.sum(z * z, 0, keepdims=True)], axis=0)


_combine_call = pl.pallas_call(
    _combine_body,
    grid=(G,),
    in_specs=[
        pl.BlockSpec((BM, HALF), lambda i: (i, 0)),
        pl.BlockSpec((BM, HALF), lambda i: (i, 0)),
        pl.BlockSpec((BM, HALF), lambda i: (i, 0)),
        pl.BlockSpec((BM, HALF), lambda i: (i, 0)),
        pl.BlockSpec((BM, 1), lambda i: (i, 0)),
        pl.BlockSpec((1, H), lambda i: (0, 0)),
    ],
    out_specs=[
        pl.BlockSpec((BM, H), lambda i: (i, 0)),
        pl.BlockSpec((2, H), lambda i: (0, 0)),
    ],
    out_shape=[
        jax.ShapeDtypeStruct((N, H), jnp.float32),
        jax.ShapeDtypeStruct((2, H), jnp.float32),
    ],
)


def _pool_body(z_ref, st_ref, g_ref, bt_ref, batch_ref, fc1w_ref, fc1b_ref,
               fc2w_ref, fc2b_ref, o_ref, pooled, cnt):
    i = pl.program_id(0)
    mean = st_ref[0:1, :] * (1.0 / N)
    var = st_ref[1:2, :] * (1.0 / N) - mean * mean
    a = g_ref[...] * lax.rsqrt(var + EPS)
    cc = bt_ref[...] - mean * a
    h = jnp.maximum(z_ref[...] * a + cc, 0.0)
    onehot = (batch_ref[...] == lax.broadcasted_iota(jnp.int32, (BM, B), 1)
              ).astype(jnp.float32)

    @pl.when(i == 0)
    def _():
        pooled[...] = jnp.zeros((B, H), jnp.float32)
        cnt[...] = jnp.zeros((B, 1), jnp.float32)

    pooled[...] += lax.dot_general(
        onehot, h, (((0,), (0,)), ((), ())), precision=lax.Precision.HIGHEST,
        preferred_element_type=jnp.float32)
    cnt[...] += lax.dot_general(
        onehot, jnp.ones((BM, 1), jnp.float32), (((0,), (0,)), ((), ())),
        precision=lax.Precision.HIGHEST, preferred_element_type=jnp.float32)

    pm = pooled[...] / jnp.maximum(cnt[...], 1.0)
    h1 = jnp.maximum(
        jnp.dot(pm, fc1w_ref[...], preferred_element_type=jnp.float32)
        + fc1b_ref[...], 0.0)
    o_ref[...] = jnp.dot(h1, fc2w_ref[...],
                         preferred_element_type=jnp.float32) + fc2b_ref[...]


_pool_call = pl.pallas_call(
    _pool_body,
    grid=(G,),
    in_specs=[
        pl.BlockSpec((BM, H), lambda i: (i, 0)),
        pl.BlockSpec((2, H), lambda i: (0, 0)),
        pl.BlockSpec((1, H), lambda i: (0, 0)),
        pl.BlockSpec((1, H), lambda i: (0, 0)),
        pl.BlockSpec((BM, 1), lambda i: (i, 0)),
        pl.BlockSpec((H, H // 2), lambda i: (0, 0)),
        pl.BlockSpec((1, H // 2), lambda i: (0, 0)),
        pl.BlockSpec((H // 2, 1), lambda i: (0, 0)),
        pl.BlockSpec((1, 1), lambda i: (0, 0)),
    ],
    out_specs=pl.BlockSpec((B, 1), lambda i: (0, 0)),
    out_shape=jax.ShapeDtypeStruct((B, 1), jnp.float32),
    scratch_shapes=[
        pltpu.VMEM((B, H), jnp.float32),
        pltpu.VMEM((B, 1), jnp.float32),
    ],
)


def kernel(x, edge_index, batch, W0, b0, gamma0, beta0, W1, b1, gamma1, beta1,
           W2, b2, gamma2, beta2, fc1_W, fc1_b, fc2_W, fc2_b):
    src = edge_index[0].astype(jnp.int32)
    dst = edge_index[1].astype(jnp.int32)
    src3 = src.reshape(NT, MSG_CHUNKS, EC)
    dst3 = dst.reshape(NT, MSG_CHUNKS, EC)
    dstd = dst.reshape(2 * NT, DEG_CHUNKS, EC)

    hist = _deg_call(dstd)
    dinv = _dinv_call(hist).reshape(NP_PAD, 1)[:N]

    Ws = [W0, W1, W2]
    bs = [b0, b1, b2]
    gs = [gamma0, gamma1, gamma2]
    bts = [beta0, beta1, beta2]

    z = None
    st = None
    for i in range(3):
        if i == 0:
            ylo, yhi = _mm1_call(x, Ws[0], dinv)
        else:
            ylo, yhi = _mm_norm_call(z, st, gs[i - 1].reshape(1, H),
                                     bts[i - 1].reshape(1, H), Ws[i], dinv)
        slo, shi = _msg_call(ylo, yhi, src3, dst3)
        z, st = _combine_call(ylo, yhi, slo, shi, dinv, bs[i].reshape(1, H))

    out = _pool_call(z, st, gs[2].reshape(1, H), bts[2].reshape(1, H),
                     batch.astype(jnp.int32).reshape(N, 1), fc1_W,
                     fc1_b.reshape(1, H // 2), fc2_W, fc2_b.reshape(1, 1))
    return out.reshape(B)


# speedup vs baseline: 1.1476x; 1.0116x over previous
import functools

import jax
import jax.numpy as jnp
from jax import lax
from jax.experimental import pallas as pl
from jax.experimental.pallas import tpu as pltpu
from jax.experimental.pallas import tpu_sc as plsc

N = 10000
E = 160000
D = 256
H = 256
B = 64
HALF = 128
NT = 16
EC = 125
NP_PAD = 10240
RPT = NP_PAD // NT
CB = 128
RB = RPT // CB
MSG_CHUNKS = E // NT // EC
SLAB = 40
DEG_CHUNKS = E // (2 * NT) // EC
SEG = NP_PAD // NT
BM = 1000
G = N // BM
EPS = 1e-5

_mesh = plsc.VectorSubcoreMesh(core_axis_name="c", subcore_axis_name="s")


def _deg_body(dst3, hist_out, idx_vm, buf, hist):
    c = lax.axis_index("c")
    s = lax.axis_index("s")
    w = c * NT + s
    pltpu.sync_copy(dst3.at[w], idx_vm)

    def _zero(j, carry):
        buf[pl.ds(j * 16, 16)] = jnp.zeros((16,), jnp.float32)
        return carry
    lax.fori_loop(0, SEG // 16, _zero, None)
    seg = s * SEG
    pltpu.sync_copy(buf, hist.at[pl.ds(seg, SEG)])
    plsc.subcore_barrier()

    def _ones(j, carry):
        buf[pl.ds(j * 16, 16)] = jnp.ones((16,), jnp.float32)
        return carry
    lax.fori_loop(0, 8, _ones, None)

    def _chunk(j, carry):
        pltpu.sync_copy(buf.at[pl.ds(0, EC)], hist.at[idx_vm.at[j]], add=True)
        return carry
    lax.fori_loop(0, DEG_CHUNKS, _chunk, None)
    plsc.subcore_barrier()
    pltpu.sync_copy(hist.at[pl.ds(seg, SEG)], buf)
    pltpu.sync_copy(buf, hist_out.at[c, pl.ds(seg, SEG)])


_deg_call = pl.kernel(
    _deg_body,
    out_type=jax.ShapeDtypeStruct((2, NP_PAD), jnp.float32),
    mesh=_mesh,
    scratch_types=[
        pltpu.VMEM((DEG_CHUNKS, EC), jnp.int32),
        pltpu.VMEM((SEG,), jnp.float32),
        pltpu.VMEM_SHARED((NP_PAD,), jnp.float32),
    ],
)


def _msg_body(y_lo, y_hi, src3, dst3, s_lo, s_hi, src_vm, dst_vm, rows0, rows1,
              acc, sem0, sem1):
    c = lax.axis_index("c")
    s = lax.axis_index("s")

    def _zrow(j, carry):
        for k in range(HALF // 16):
            rows0[j, pl.ds(k * 16, 16)] = jnp.zeros((16,), jnp.float32)
        return carry
    lax.fori_loop(0, CB, _zrow, None)
    base = s * RPT
    for k in range(RB):
        pltpu.async_copy(rows0, acc.at[pl.ds(base + k * CB, CB)], sem1)
    pltpu.sync_copy(src3.at[s, pl.ds(0, SLAB)], src_vm)
    pltpu.sync_copy(dst3.at[s, pl.ds(0, SLAB)], dst_vm)
    for k in range(RB):
        pltpu.make_async_copy(rows0, acc.at[pl.ds(base + k * CB, CB)], sem1).wait()
    plsc.subcore_barrier()

    bufs = (rows0.at[pl.ds(0, EC)], rows1.at[pl.ds(0, EC)])
    sems = (sem0, sem1)

    def _pass(tbl, out):
        def _gather(ch, b):
            pltpu.async_copy(tbl.at[src_vm.at[ch]], bufs[b], sems[b])

        def _step(ch, b, issue_next):
            pltpu.make_async_copy(tbl.at[src_vm.at[ch]], bufs[b], sems[b]).wait()
            pltpu.sync_copy(bufs[b], acc.at[dst_vm.at[ch]], add=True)
            if issue_next:
                _gather(ch + 2, b)

        for h in range(MSG_CHUNKS // SLAB):
            if h > 0:
                pltpu.sync_copy(src3.at[s, pl.ds(h * SLAB, SLAB)], src_vm)
                pltpu.sync_copy(dst3.at[s, pl.ds(h * SLAB, SLAB)], dst_vm)
            _gather(0, 0)
            _gather(1, 1)

            def _loop(j, carry):
                _step(2 * j, 0, True)
                _step(2 * j + 1, 1, True)
                return carry
            lax.fori_loop(0, SLAB // 2 - 1, _loop, None)
            _step(SLAB - 2, 0, False)
            _step(SLAB - 1, 1, False)

        plsc.subcore_barrier()
        pltpu.sync_copy(acc.at[pl.ds(base, RPT)], out.at[pl.ds(base, RPT)])

    pl.when(c == 0)(lambda: _pass(y_lo, s_lo))
    pl.when(c == 1)(lambda: _pass(y_hi, s_hi))


_msg_call = pl.kernel(
    _msg_body,
    out_type=[jax.ShapeDtypeStruct((NP_PAD, HALF), jnp.float32)] * 2,
    mesh=_mesh,
    scratch_types=[
        pltpu.VMEM((SLAB, EC), jnp.int32),
        pltpu.VMEM((SLAB, EC), jnp.int32),
        pltpu.VMEM((CB, HALF), jnp.float32),
        pltpu.VMEM((EC, HALF), jnp.float32),
        pltpu.VMEM_SHARED((NP_PAD, HALF), jnp.float32),
        pltpu.SemaphoreType.DMA,
        pltpu.SemaphoreType.DMA,
    ],
)


def _dinv_body(h_ref, o_ref):
    deg = h_ref[0:1, :] + h_ref[1:2, :] + 1.0
    o_ref[...] = lax.rsqrt(deg)


_dinv_call = pl.pallas_call(
    _dinv_body,
    out_shape=jax.ShapeDtypeStruct((1, NP_PAD), jnp.float32),
)


def _mm1_body(x_ref, w_ref, dinv_ref, ylo_ref, yhi_ref):
    t = jnp.dot(x_ref[...], w_ref[...], preferred_element_type=jnp.float32)
    y = t * dinv_ref[...]
    ylo_ref[...] = y[:, :HALF]
    yhi_ref[...] = y[:, HALF:]


_mm1_call = pl.pallas_call(
    _mm1_body,
    grid=(G,),
    in_specs=[
        pl.BlockSpec((BM, D), lambda i: (i, 0)),
        pl.BlockSpec((D, H), lambda i: (0, 0)),
        pl.BlockSpec((BM, 1), lambda i: (i, 0)),
    ],
    out_specs=[pl.BlockSpec((BM, HALF), lambda i: (i, 0))] * 2,
    out_shape=[jax.ShapeDtypeStruct((N, HALF), jnp.float32)] * 2,
)


def _mm_norm_body(z_ref, st_ref, g_ref, bt_ref, w_ref, dinv_ref, ylo_ref, yhi_ref):
    mean = st_ref[0:1, :] * (1.0 / N)
    var = st_ref[1:2, :] * (1.0 / N) - mean * mean
    a = g_ref[...] * lax.rsqrt(var + EPS)
    cc = bt_ref[...] - mean * a
    h = jnp.maximum(z_ref[...] * a + cc, 0.0)
    t = jnp.dot(h, w_ref[...], preferred_element_type=jnp.float32)
    y = t * dinv_ref[...]
    ylo_ref[...] = y[:, :HALF]
    yhi_ref[...] = y[:, HALF:]


_mm_norm_call = pl.pallas_call(
    _mm_norm_body,
    grid=(G,),
    in_specs=[
        pl.BlockSpec((BM, H), lambda i: (i, 0)),
        pl.BlockSpec((2, H), lambda i: (0, 0)),
        pl.BlockSpec((1, H), lambda i: (0, 0)),
        pl.BlockSpec((1, H), lambda i: (0, 0)),
        pl.BlockSpec((H, H), lambda i: (0, 0)),
        pl.BlockSpec((BM, 1), lambda i: (i, 0)),
    ],
    out_specs=[pl.BlockSpec((BM, HALF), lambda i: (i, 0))] * 2,
    out_shape=[jax.ShapeDtypeStruct((N, HALF), jnp.float32)] * 2,
)


def _combine_body(ylo_ref, yhi_ref, slo_ref, shi_ref, dinv_ref, b_ref, z_ref, st_ref):
    i = pl.program_id(0)
    d = dinv_ref[...]
    zlo = d * (slo_ref[...] + ylo_ref[...]) + b_ref[:, :HALF]
    zhi = d * (shi_ref[...] + yhi_ref[...]) + b_ref[:, HALF:]
    z = jnp.concatenate([zlo, zhi], axis=1)
    z_ref[...] = z

    @pl.when(i == 0)
    def _():
        st_ref[...] = jnp.zeros((2, H), jnp.float32)

    st_ref[...] += jnp.concatenate(
        [jnp.sum(z, 0, keepdims=True), jnp.sum(z * z, 0, keepdims=True)], axis=0)


_combine_call = pl.pallas_call(
    _combine_body,
    grid=(G,),
    in_specs=[
        pl.BlockSpec((BM, HALF), lambda i: (i, 0)),
        pl.BlockSpec((BM, HALF), lambda i: (i, 0)),
        pl.BlockSpec((BM, HALF), lambda i: (i, 0)),
        pl.BlockSpec((BM, HALF), lambda i: (i, 0)),
        pl.BlockSpec((BM, 1), lambda i: (i, 0)),
        pl.BlockSpec((1, H), lambda i: (0, 0)),
    ],
    out_specs=[
        pl.BlockSpec((BM, H), lambda i: (i, 0)),
        pl.BlockSpec((2, H), lambda i: (0, 0)),
    ],
    out_shape=[
        jax.ShapeDtypeStruct((N, H), jnp.float32),
        jax.ShapeDtypeStruct((2, H), jnp.float32),
    ],
)


def _pool_body(z_ref, st_ref, g_ref, bt_ref, batch_ref, fc1w_ref, fc1b_ref,
               fc2w_ref, fc2b_ref, o_ref, pooled, cnt):
    i = pl.program_id(0)
    mean = st_ref[0:1, :] * (1.0 / N)
    var = st_ref[1:2, :] * (1.0 / N) - mean * mean
    a = g_ref[...] * lax.rsqrt(var + EPS)
    cc = bt_ref[...] - mean * a
    h = jnp.maximum(z_ref[...] * a + cc, 0.0)
    onehot = (batch_ref[...] == lax.broadcasted_iota(jnp.int32, (BM, B), 1)
              ).astype(jnp.float32)

    @pl.when(i == 0)
    def _():
        pooled[...] = jnp.zeros((B, H), jnp.float32)
        cnt[...] = jnp.zeros((B, 1), jnp.float32)

    pooled[...] += lax.dot_general(
        onehot, h, (((0,), (0,)), ((), ())), precision=lax.Precision.HIGHEST,
        preferred_element_type=jnp.float32)
    cnt[...] += lax.dot_general(
        onehot, jnp.ones((BM, 1), jnp.float32), (((0,), (0,)), ((), ())),
        precision=lax.Precision.HIGHEST, preferred_element_type=jnp.float32)

    pm = pooled[...] / jnp.maximum(cnt[...], 1.0)
    h1 = jnp.maximum(
        jnp.dot(pm, fc1w_ref[...], preferred_element_type=jnp.float32)
        + fc1b_ref[...], 0.0)
    o_ref[...] = jnp.dot(h1, fc2w_ref[...],
                         preferred_element_type=jnp.float32) + fc2b_ref[...]


_pool_call = pl.pallas_call(
    _pool_body,
    grid=(G,),
    in_specs=[
        pl.BlockSpec((BM, H), lambda i: (i, 0)),
        pl.BlockSpec((2, H), lambda i: (0, 0)),
        pl.BlockSpec((1, H), lambda i: (0, 0)),
        pl.BlockSpec((1, H), lambda i: (0, 0)),
        pl.BlockSpec((BM, 1), lambda i: (i, 0)),
        pl.BlockSpec((H, H // 2), lambda i: (0, 0)),
        pl.BlockSpec((1, H // 2), lambda i: (0, 0)),
        pl.BlockSpec((H // 2, 1), lambda i: (0, 0)),
        pl.BlockSpec((1, 1), lambda i: (0, 0)),
    ],
    out_specs=pl.BlockSpec((B, 1), lambda i: (0, 0)),
    out_shape=jax.ShapeDtypeStruct((B, 1), jnp.float32),
    scratch_shapes=[
        pltpu.VMEM((B, H), jnp.float32),
        pltpu.VMEM((B, 1), jnp.float32),
    ],
)


def kernel(x, edge_index, batch, W0, b0, gamma0, beta0, W1, b1, gamma1, beta1,
           W2, b2, gamma2, beta2, fc1_W, fc1_b, fc2_W, fc2_b):
    src = edge_index[0].astype(jnp.int32)
    dst = edge_index[1].astype(jnp.int32)
    src3 = src.reshape(NT, MSG_CHUNKS, EC)
    dst3 = dst.reshape(NT, MSG_CHUNKS, EC)
    dstd = dst.reshape(2 * NT, DEG_CHUNKS, EC)

    hist = _deg_call(dstd)
    dinv = _dinv_call(hist).reshape(NP_PAD, 1)[:N]

    Ws = [W0, W1, W2]
    bs = [b0, b1, b2]
    gs = [gamma0, gamma1, gamma2]
    bts = [beta0, beta1, beta2]

    z = None
    st = None
    for i in range(3):
        if i == 0:
            ylo, yhi = _mm1_call(x, Ws[0], dinv)
        else:
            ylo, yhi = _mm_norm_call(z, st, gs[i - 1].reshape(1, H),
                                     bts[i - 1].reshape(1, H), Ws[i], dinv)
        slo, shi = _msg_call(ylo, yhi, src3, dst3)
        z, st = _combine_call(ylo, yhi, slo, shi, dinv, bs[i].reshape(1, H))

    out = _pool_call(z, st, gs[2].reshape(1, H), bts[2].reshape(1, H),
                     batch.astype(jnp.int32).reshape(N, 1), fc1_W,
                     fc1_b.reshape(1, H // 2), fc2_W, fc2_b.reshape(1, 1))
    return out.reshape(B)


# speedup vs baseline: 1.1798x; 1.0281x over previous
import functools

import jax
import jax.numpy as jnp
from jax import lax
from jax.experimental import pallas as pl
from jax.experimental.pallas import tpu as pltpu
from jax.experimental.pallas import tpu_sc as plsc

N = 10000
E = 160000
D = 256
H = 256
B = 64
HALF = 128
NT = 16
EC = 125
NP_PAD = 10240
RPT = NP_PAD // NT
CB = 128
RB = RPT // CB
MSG_CHUNKS = E // NT // EC
SLAB = 40
DEG_CHUNKS = E // (2 * NT) // EC
SEG = NP_PAD // NT
BM = 2000
G = N // BM
EPS = 1e-5

_mesh = plsc.VectorSubcoreMesh(core_axis_name="c", subcore_axis_name="s")


def _deg_body(dst3, hist_out, idx_vm, buf, hist):
    c = lax.axis_index("c")
    s = lax.axis_index("s")
    w = c * NT + s
    pltpu.sync_copy(dst3.at[w], idx_vm)

    def _zero(j, carry):
        buf[pl.ds(j * 16, 16)] = jnp.zeros((16,), jnp.float32)
        return carry
    lax.fori_loop(0, SEG // 16, _zero, None)
    seg = s * SEG
    pltpu.sync_copy(buf, hist.at[pl.ds(seg, SEG)])
    plsc.subcore_barrier()

    def _ones(j, carry):
        buf[pl.ds(j * 16, 16)] = jnp.ones((16,), jnp.float32)
        return carry
    lax.fori_loop(0, 8, _ones, None)

    def _chunk(j, carry):
        pltpu.sync_copy(buf.at[pl.ds(0, EC)], hist.at[idx_vm.at[j]], add=True)
        return carry
    lax.fori_loop(0, DEG_CHUNKS, _chunk, None)
    plsc.subcore_barrier()
    pltpu.sync_copy(hist.at[pl.ds(seg, SEG)], buf)
    pltpu.sync_copy(buf, hist_out.at[c, pl.ds(seg, SEG)])


_deg_call = pl.kernel(
    _deg_body,
    out_type=jax.ShapeDtypeStruct((2, NP_PAD), jnp.float32),
    mesh=_mesh,
    scratch_types=[
        pltpu.VMEM((DEG_CHUNKS, EC), jnp.int32),
        pltpu.VMEM((SEG,), jnp.float32),
        pltpu.VMEM_SHARED((NP_PAD,), jnp.float32),
    ],
)


def _msg_body(y_lo, y_hi, src3, dst3, s_lo, s_hi, src_vm, dst_vm, rows0, rows1,
              acc, sem0, sem1):
    c = lax.axis_index("c")
    s = lax.axis_index("s")

    def _zrow(j, carry):
        for k in range(HALF // 16):
            rows0[j, pl.ds(k * 16, 16)] = jnp.zeros((16,), jnp.float32)
        return carry
    lax.fori_loop(0, CB, _zrow, None)
    base = s * RPT
    for k in range(RB):
        pltpu.async_copy(rows0, acc.at[pl.ds(base + k * CB, CB)], sem1)
    pltpu.sync_copy(src3.at[s, pl.ds(0, SLAB)], src_vm)
    pltpu.sync_copy(dst3.at[s, pl.ds(0, SLAB)], dst_vm)
    for k in range(RB):
        pltpu.make_async_copy(rows0, acc.at[pl.ds(base + k * CB, CB)], sem1).wait()
    plsc.subcore_barrier()

    bufs = (rows0.at[pl.ds(0, EC)], rows1.at[pl.ds(0, EC)])
    sems = (sem0, sem1)

    def _pass(tbl, out):
        def _gather(ch, b):
            pltpu.async_copy(tbl.at[src_vm.at[ch]], bufs[b], sems[b])

        def _step(ch, b, issue_next):
            pltpu.make_async_copy(tbl.at[src_vm.at[ch]], bufs[b], sems[b]).wait()
            pltpu.sync_copy(bufs[b], acc.at[dst_vm.at[ch]], add=True)
            if issue_next:
                _gather(ch + 2, b)

        for h in range(MSG_CHUNKS // SLAB):
            if h > 0:
                pltpu.sync_copy(src3.at[s, pl.ds(h * SLAB, SLAB)], src_vm)
                pltpu.sync_copy(dst3.at[s, pl.ds(h * SLAB, SLAB)], dst_vm)
            _gather(0, 0)
            _gather(1, 1)

            def _loop(j, carry):
                _step(2 * j, 0, True)
                _step(2 * j + 1, 1, True)
                return carry
            lax.fori_loop(0, SLAB // 2 - 1, _loop, None)
            _step(SLAB - 2, 0, False)
            _step(SLAB - 1, 1, False)

        plsc.subcore_barrier()
        pltpu.sync_copy(acc.at[pl.ds(base, RPT)], out.at[pl.ds(base, RPT)])

    pl.when(c == 0)(lambda: _pass(y_lo, s_lo))
    pl.when(c == 1)(lambda: _pass(y_hi, s_hi))


_msg_call = pl.kernel(
    _msg_body,
    out_type=[jax.ShapeDtypeStruct((NP_PAD, HALF), jnp.float32)] * 2,
    mesh=_mesh,
    scratch_types=[
        pltpu.VMEM((SLAB, EC), jnp.int32),
        pltpu.VMEM((SLAB, EC), jnp.int32),
        pltpu.VMEM((CB, HALF), jnp.float32),
        pltpu.VMEM((EC, HALF), jnp.float32),
        pltpu.VMEM_SHARED((NP_PAD, HALF), jnp.float32),
        pltpu.SemaphoreType.DMA,
        pltpu.SemaphoreType.DMA,
    ],
)


def _dinv_body(h_ref, o_ref):
    deg = h_ref[0:1, :] + h_ref[1:2, :] + 1.0
    o_ref[...] = lax.rsqrt(deg)


_dinv_call = pl.pallas_call(
    _dinv_body,
    out_shape=jax.ShapeDtypeStruct((1, NP_PAD), jnp.float32),
)


def _mm1_body(x_ref, w_ref, dinv_ref, ylo_ref, yhi_ref):
    t = jnp.dot(x_ref[...], w_ref[...], preferred_element_type=jnp.float32)
    y = t * dinv_ref[...]
    ylo_ref[...] = y[:, :HALF]
    yhi_ref[...] = y[:, HALF:]


_mm1_call = pl.pallas_call(
    _mm1_body,
    grid=(G,),
    in_specs=[
        pl.BlockSpec((BM, D), lambda i: (i, 0)),
        pl.BlockSpec((D, H), lambda i: (0, 0)),
        pl.BlockSpec((BM, 1), lambda i: (i, 0)),
    ],
    out_specs=[pl.BlockSpec((BM, HALF), lambda i: (i, 0))] * 2,
    out_shape=[jax.ShapeDtypeStruct((N, HALF), jnp.float32)] * 2,
)


def _mm_norm_body(z_ref, st_ref, g_ref, bt_ref, w_ref, dinv_ref, ylo_ref, yhi_ref):
    mean = st_ref[0:1, :] * (1.0 / N)
    var = st_ref[1:2, :] * (1.0 / N) - mean * mean
    a = g_ref[...] * lax.rsqrt(var + EPS)
    cc = bt_ref[...] - mean * a
    h = jnp.maximum(z_ref[...] * a + cc, 0.0)
    t = jnp.dot(h, w_ref[...], preferred_element_type=jnp.float32)
    y = t * dinv_ref[...]
    ylo_ref[...] = y[:, :HALF]
    yhi_ref[...] = y[:, HALF:]


_mm_norm_call = pl.pallas_call(
    _mm_norm_body,
    grid=(G,),
    in_specs=[
        pl.BlockSpec((BM, H), lambda i: (i, 0)),
        pl.BlockSpec((2, H), lambda i: (0, 0)),
        pl.BlockSpec((1, H), lambda i: (0, 0)),
        pl.BlockSpec((1, H), lambda i: (0, 0)),
        pl.BlockSpec((H, H), lambda i: (0, 0)),
        pl.BlockSpec((BM, 1), lambda i: (i, 0)),
    ],
    out_specs=[pl.BlockSpec((BM, HALF), lambda i: (i, 0))] * 2,
    out_shape=[jax.ShapeDtypeStruct((N, HALF), jnp.float32)] * 2,
)


def _combine_body(ylo_ref, yhi_ref, slo_ref, shi_ref, dinv_ref, b_ref, z_ref, st_ref):
    i = pl.program_id(0)
    d = dinv_ref[...]
    zlo = d * (slo_ref[...] + ylo_ref[...]) + b_ref[:, :HALF]
    zhi = d * (shi_ref[...] + yhi_ref[...]) + b_ref[:, HALF:]
    z = jnp.concatenate([zlo, zhi], axis=1)
    z_ref[...] = z

    @pl.when(i == 0)
    def _():
        st_ref[...] = jnp.zeros((2, H), jnp.float32)

    st_ref[...] += jnp.concatenate(
        [jnp.sum(z, 0, keepdims=True), jnp.sum(z * z, 0, keepdims=True)], axis=0)


_combine_call = pl.pallas_call(
    _combine_body,
    grid=(G,),
    in_specs=[
        pl.BlockSpec((BM, HALF), lambda i: (i, 0)),
        pl.BlockSpec((BM, HALF), lambda i: (i, 0)),
        pl.BlockSpec((BM, HALF), lambda i: (i, 0)),
        pl.BlockSpec((BM, HALF), lambda i: (i, 0)),
        pl.BlockSpec((BM, 1), lambda i: (i, 0)),
        pl.BlockSpec((1, H), lambda i: (0, 0)),
    ],
    out_specs=[
        pl.BlockSpec((BM, H), lambda i: (i, 0)),
        pl.BlockSpec((2, H), lambda i: (0, 0)),
    ],
    out_shape=[
        jax.ShapeDtypeStruct((N, H), jnp.float32),
        jax.ShapeDtypeStruct((2, H), jnp.float32),
    ],
)


def _pool_body(z_ref, st_ref, g_ref, bt_ref, batch_ref, fc1w_ref, fc1b_ref,
               fc2w_ref, fc2b_ref, o_ref, pooled, cnt):
    i = pl.program_id(0)
    mean = st_ref[0:1, :] * (1.0 / N)
    var = st_ref[1:2, :] * (1.0 / N) - mean * mean
    a = g_ref[...] * lax.rsqrt(var + EPS)
    cc = bt_ref[...] - mean * a
    h = jnp.maximum(z_ref[...] * a + cc, 0.0)
    onehot = (batch_ref[...] == lax.broadcasted_iota(jnp.int32, (BM, B), 1)
              ).astype(jnp.float32)

    @pl.when(i == 0)
    def _():
        pooled[...] = jnp.zeros((B, H), jnp.float32)
        cnt[...] = jnp.zeros((B, 1), jnp.float32)

    pooled[...] += lax.dot_general(
        onehot, h, (((0,), (0,)), ((), ())), precision=lax.Precision.HIGHEST,
        preferred_element_type=jnp.float32)
    cnt[...] += lax.dot_general(
        onehot, jnp.ones((BM, 1), jnp.float32), (((0,), (0,)), ((), ())),
        precision=lax.Precision.HIGHEST, preferred_element_type=jnp.float32)

    pm = pooled[...] / jnp.maximum(cnt[...], 1.0)
    h1 = jnp.maximum(
        jnp.dot(pm, fc1w_ref[...], preferred_element_type=jnp.float32)
        + fc1b_ref[...], 0.0)
    o_ref[...] = jnp.dot(h1, fc2w_ref[...],
                         preferred_element_type=jnp.float32) + fc2b_ref[...]


_pool_call = pl.pallas_call(
    _pool_body,
    grid=(G,),
    in_specs=[
        pl.BlockSpec((BM, H), lambda i: (i, 0)),
        pl.BlockSpec((2, H), lambda i: (0, 0)),
        pl.BlockSpec((1, H), lambda i: (0, 0)),
        pl.BlockSpec((1, H), lambda i: (0, 0)),
        pl.BlockSpec((BM, 1), lambda i: (i, 0)),
        pl.BlockSpec((H, H // 2), lambda i: (0, 0)),
        pl.BlockSpec((1, H // 2), lambda i: (0, 0)),
        pl.BlockSpec((H // 2, 1), lambda i: (0, 0)),
        pl.BlockSpec((1, 1), lambda i: (0, 0)),
    ],
    out_specs=pl.BlockSpec((B, 1), lambda i: (0, 0)),
    out_shape=jax.ShapeDtypeStruct((B, 1), jnp.float32),
    scratch_shapes=[
        pltpu.VMEM((B, H), jnp.float32),
        pltpu.VMEM((B, 1), jnp.float32),
    ],
)


def kernel(x, edge_index, batch, W0, b0, gamma0, beta0, W1, b1, gamma1, beta1,
           W2, b2, gamma2, beta2, fc1_W, fc1_b, fc2_W, fc2_b):
    src = edge_index[0].astype(jnp.int32)
    dst = edge_index[1].astype(jnp.int32)
    src3 = src.reshape(NT, MSG_CHUNKS, EC)
    dst3 = dst.reshape(NT, MSG_CHUNKS, EC)
    dstd = dst.reshape(2 * NT, DEG_CHUNKS, EC)

    hist = _deg_call(dstd)
    dinv = _dinv_call(hist).reshape(NP_PAD, 1)[:N]

    Ws = [W0, W1, W2]
    bs = [b0, b1, b2]
    gs = [gamma0, gamma1, gamma2]
    bts = [beta0, beta1, beta2]

    z = None
    st = None
    for i in range(3):
        if i == 0:
            ylo, yhi = _mm1_call(x, Ws[0], dinv)
        else:
            ylo, yhi = _mm_norm_call(z, st, gs[i - 1].reshape(1, H),
                                     bts[i - 1].reshape(1, H), Ws[i], dinv)
        slo, shi = _msg_call(ylo, yhi, src3, dst3)
        z, st = _combine_call(ylo, yhi, slo, shi, dinv, bs[i].reshape(1, H))

    out = _pool_call(z, st, gs[2].reshape(1, H), bts[2].reshape(1, H),
                     batch.astype(jnp.int32).reshape(N, 1), fc1_W,
                     fc1_b.reshape(1, H // 2), fc2_W, fc2_b.reshape(1, 1))
    return out.reshape(B)


# speedup vs baseline: 1.1976x; 1.0150x over previous
import functools

import jax
import jax.numpy as jnp
from jax import lax
from jax.experimental import pallas as pl
from jax.experimental.pallas import tpu as pltpu
from jax.experimental.pallas import tpu_sc as plsc

N = 10000
E = 160000
D = 256
H = 256
B = 64
HALF = 128
NT = 16
EC = 125
NP_PAD = 10240
RPT = NP_PAD // NT
CB = 128
RB = RPT // CB
MSG_CHUNKS = E // NT // EC
SLAB = 40
DEG_CHUNKS = E // (2 * NT) // EC
SEG = NP_PAD // NT
BM = 5000
G = N // BM
EPS = 1e-5

_mesh = plsc.VectorSubcoreMesh(core_axis_name="c", subcore_axis_name="s")


def _deg_body(dst3, hist_out, idx_vm, buf, hist):
    c = lax.axis_index("c")
    s = lax.axis_index("s")
    w = c * NT + s
    pltpu.sync_copy(dst3.at[w], idx_vm)

    def _zero(j, carry):
        buf[pl.ds(j * 16, 16)] = jnp.zeros((16,), jnp.float32)
        return carry
    lax.fori_loop(0, SEG // 16, _zero, None)
    seg = s * SEG
    pltpu.sync_copy(buf, hist.at[pl.ds(seg, SEG)])
    plsc.subcore_barrier()

    def _ones(j, carry):
        buf[pl.ds(j * 16, 16)] = jnp.ones((16,), jnp.float32)
        return carry
    lax.fori_loop(0, 8, _ones, None)

    def _chunk(j, carry):
        pltpu.sync_copy(buf.at[pl.ds(0, EC)], hist.at[idx_vm.at[j]], add=True)
        return carry
    lax.fori_loop(0, DEG_CHUNKS, _chunk, None)
    plsc.subcore_barrier()
    pltpu.sync_copy(hist.at[pl.ds(seg, SEG)], buf)
    pltpu.sync_copy(buf, hist_out.at[c, pl.ds(seg, SEG)])


_deg_call = pl.kernel(
    _deg_body,
    out_type=jax.ShapeDtypeStruct((2, NP_PAD), jnp.float32),
    mesh=_mesh,
    scratch_types=[
        pltpu.VMEM((DEG_CHUNKS, EC), jnp.int32),
        pltpu.VMEM((SEG,), jnp.float32),
        pltpu.VMEM_SHARED((NP_PAD,), jnp.float32),
    ],
)


def _msg_body(y_lo, y_hi, src3, dst3, s_lo, s_hi, src_vm, dst_vm, rows0, rows1,
              acc, sem0, sem1):
    c = lax.axis_index("c")
    s = lax.axis_index("s")

    def _zrow(j, carry):
        for k in range(HALF // 16):
            rows0[j, pl.ds(k * 16, 16)] = jnp.zeros((16,), jnp.float32)
        return carry
    lax.fori_loop(0, CB, _zrow, None)
    base = s * RPT
    for k in range(RB):
        pltpu.async_copy(rows0, acc.at[pl.ds(base + k * CB, CB)], sem1)
    pltpu.sync_copy(src3.at[s, pl.ds(0, SLAB)], src_vm)
    pltpu.sync_copy(dst3.at[s, pl.ds(0, SLAB)], dst_vm)
    for k in range(RB):
        pltpu.make_async_copy(rows0, acc.at[pl.ds(base + k * CB, CB)], sem1).wait()
    plsc.subcore_barrier()

    bufs = (rows0.at[pl.ds(0, EC)], rows1.at[pl.ds(0, EC)])
    sems = (sem0, sem1)

    def _pass(tbl, out):
        def _gather(ch, b):
            pltpu.async_copy(tbl.at[src_vm.at[ch]], bufs[b], sems[b])

        def _step(ch, b, issue_next):
            pltpu.make_async_copy(tbl.at[src_vm.at[ch]], bufs[b], sems[b]).wait()
            pltpu.sync_copy(bufs[b], acc.at[dst_vm.at[ch]], add=True)
            if issue_next:
                _gather(ch + 2, b)

        for h in range(MSG_CHUNKS // SLAB):
            if h > 0:
                pltpu.sync_copy(src3.at[s, pl.ds(h * SLAB, SLAB)], src_vm)
                pltpu.sync_copy(dst3.at[s, pl.ds(h * SLAB, SLAB)], dst_vm)
            _gather(0, 0)
            _gather(1, 1)

            def _loop(j, carry):
                _step(2 * j, 0, True)
                _step(2 * j + 1, 1, True)
                return carry
            lax.fori_loop(0, SLAB // 2 - 1, _loop, None)
            _step(SLAB - 2, 0, False)
            _step(SLAB - 1, 1, False)

        plsc.subcore_barrier()
        pltpu.sync_copy(acc.at[pl.ds(base, RPT)], out.at[pl.ds(base, RPT)])

    pl.when(c == 0)(lambda: _pass(y_lo, s_lo))
    pl.when(c == 1)(lambda: _pass(y_hi, s_hi))


_msg_call = pl.kernel(
    _msg_body,
    out_type=[jax.ShapeDtypeStruct((NP_PAD, HALF), jnp.float32)] * 2,
    mesh=_mesh,
    scratch_types=[
        pltpu.VMEM((SLAB, EC), jnp.int32),
        pltpu.VMEM((SLAB, EC), jnp.int32),
        pltpu.VMEM((CB, HALF), jnp.float32),
        pltpu.VMEM((EC, HALF), jnp.float32),
        pltpu.VMEM_SHARED((NP_PAD, HALF), jnp.float32),
        pltpu.SemaphoreType.DMA,
        pltpu.SemaphoreType.DMA,
    ],
)


def _dinv_body(h_ref, o_ref):
    deg = h_ref[0:1, :] + h_ref[1:2, :] + 1.0
    o_ref[...] = lax.rsqrt(deg)


_dinv_call = pl.pallas_call(
    _dinv_body,
    out_shape=jax.ShapeDtypeStruct((1, NP_PAD), jnp.float32),
)


def _mm1_body(x_ref, w_ref, dinv_ref, ylo_ref, yhi_ref):
    t = jnp.dot(x_ref[...], w_ref[...], preferred_element_type=jnp.float32)
    y = t * dinv_ref[...]
    ylo_ref[...] = y[:, :HALF]
    yhi_ref[...] = y[:, HALF:]


_mm1_call = pl.pallas_call(
    _mm1_body,
    grid=(G,),
    in_specs=[
        pl.BlockSpec((BM, D), lambda i: (i, 0)),
        pl.BlockSpec((D, H), lambda i: (0, 0)),
        pl.BlockSpec((BM, 1), lambda i: (i, 0)),
    ],
    out_specs=[pl.BlockSpec((BM, HALF), lambda i: (i, 0))] * 2,
    out_shape=[jax.ShapeDtypeStruct((N, HALF), jnp.float32)] * 2,
)


def _mm_norm_body(z_ref, st_ref, g_ref, bt_ref, w_ref, dinv_ref, ylo_ref, yhi_ref):
    mean = st_ref[0:1, :] * (1.0 / N)
    var = st_ref[1:2, :] * (1.0 / N) - mean * mean
    a = g_ref[...] * lax.rsqrt(var + EPS)
    cc = bt_ref[...] - mean * a
    h = jnp.maximum(z_ref[...] * a + cc, 0.0)
    t = jnp.dot(h, w_ref[...], preferred_element_type=jnp.float32)
    y = t * dinv_ref[...]
    ylo_ref[...] = y[:, :HALF]
    yhi_ref[...] = y[:, HALF:]


_mm_norm_call = pl.pallas_call(
    _mm_norm_body,
    grid=(G,),
    in_specs=[
        pl.BlockSpec((BM, H), lambda i: (i, 0)),
        pl.BlockSpec((2, H), lambda i: (0, 0)),
        pl.BlockSpec((1, H), lambda i: (0, 0)),
        pl.BlockSpec((1, H), lambda i: (0, 0)),
        pl.BlockSpec((H, H), lambda i: (0, 0)),
        pl.BlockSpec((BM, 1), lambda i: (i, 0)),
    ],
    out_specs=[pl.BlockSpec((BM, HALF), lambda i: (i, 0))] * 2,
    out_shape=[jax.ShapeDtypeStruct((N, HALF), jnp.float32)] * 2,
)


def _combine_body(ylo_ref, yhi_ref, slo_ref, shi_ref, dinv_ref, b_ref, z_ref, st_ref):
    i = pl.program_id(0)
    d = dinv_ref[...]
    zlo = d * (slo_ref[...] + ylo_ref[...]) + b_ref[:, :HALF]
    zhi = d * (shi_ref[...] + yhi_ref[...]) + b_ref[:, HALF:]
    z = jnp.concatenate([zlo, zhi], axis=1)
    z_ref[...] = z

    @pl.when(i == 0)
    def _():
        st_ref[...] = jnp.zeros((2, H), jnp.float32)

    st_ref[...] += jnp.concatenate(
        [jnp.sum(z, 0, keepdims=True), jnp.sum(z * z, 0, keepdims=True)], axis=0)


_combine_call = pl.pallas_call(
    _combine_body,
    grid=(G,),
    in_specs=[
        pl.BlockSpec((BM, HALF), lambda i: (i, 0)),
        pl.BlockSpec((BM, HALF), lambda i: (i, 0)),
        pl.BlockSpec((BM, HALF), lambda i: (i, 0)),
        pl.BlockSpec((BM, HALF), lambda i: (i, 0)),
        pl.BlockSpec((BM, 1), lambda i: (i, 0)),
        pl.BlockSpec((1, H), lambda i: (0, 0)),
    ],
    out_specs=[
        pl.BlockSpec((BM, H), lambda i: (i, 0)),
        pl.BlockSpec((2, H), lambda i: (0, 0)),
    ],
    out_shape=[
        jax.ShapeDtypeStruct((N, H), jnp.float32),
        jax.ShapeDtypeStruct((2, H), jnp.float32),
    ],
)


def _pool_body(z_ref, st_ref, g_ref, bt_ref, batch_ref, fc1w_ref, fc1b_ref,
               fc2w_ref, fc2b_ref, o_ref, pooled, cnt):
    i = pl.program_id(0)
    mean = st_ref[0:1, :] * (1.0 / N)
    var = st_ref[1:2, :] * (1.0 / N) - mean * mean
    a = g_ref[...] * lax.rsqrt(var + EPS)
    cc = bt_ref[...] - mean * a
    h = jnp.maximum(z_ref[...] * a + cc, 0.0)
    onehot = (batch_ref[...] == lax.broadcasted_iota(jnp.int32, (BM, B), 1)
              ).astype(jnp.float32)

    @pl.when(i == 0)
    def _():
        pooled[...] = jnp.zeros((B, H), jnp.float32)
        cnt[...] = jnp.zeros((B, 1), jnp.float32)

    pooled[...] += lax.dot_general(
        onehot, h, (((0,), (0,)), ((), ())), precision=lax.Precision.HIGHEST,
        preferred_element_type=jnp.float32)
    cnt[...] += lax.dot_general(
        onehot, jnp.ones((BM, 1), jnp.float32), (((0,), (0,)), ((), ())),
        precision=lax.Precision.HIGHEST, preferred_element_type=jnp.float32)

    pm = pooled[...] / jnp.maximum(cnt[...], 1.0)
    h1 = jnp.maximum(
        jnp.dot(pm, fc1w_ref[...], preferred_element_type=jnp.float32)
        + fc1b_ref[...], 0.0)
    o_ref[...] = jnp.dot(h1, fc2w_ref[...],
                         preferred_element_type=jnp.float32) + fc2b_ref[...]


_pool_call = pl.pallas_call(
    _pool_body,
    grid=(G,),
    in_specs=[
        pl.BlockSpec((BM, H), lambda i: (i, 0)),
        pl.BlockSpec((2, H), lambda i: (0, 0)),
        pl.BlockSpec((1, H), lambda i: (0, 0)),
        pl.BlockSpec((1, H), lambda i: (0, 0)),
        pl.BlockSpec((BM, 1), lambda i: (i, 0)),
        pl.BlockSpec((H, H // 2), lambda i: (0, 0)),
        pl.BlockSpec((1, H // 2), lambda i: (0, 0)),
        pl.BlockSpec((H // 2, 1), lambda i: (0, 0)),
        pl.BlockSpec((1, 1), lambda i: (0, 0)),
    ],
    out_specs=pl.BlockSpec((B, 1), lambda i: (0, 0)),
    out_shape=jax.ShapeDtypeStruct((B, 1), jnp.float32),
    scratch_shapes=[
        pltpu.VMEM((B, H), jnp.float32),
        pltpu.VMEM((B, 1), jnp.float32),
    ],
)


def kernel(x, edge_index, batch, W0, b0, gamma0, beta0, W1, b1, gamma1, beta1,
           W2, b2, gamma2, beta2, fc1_W, fc1_b, fc2_W, fc2_b):
    src = edge_index[0].astype(jnp.int32)
    dst = edge_index[1].astype(jnp.int32)
    src3 = src.reshape(NT, MSG_CHUNKS, EC)
    dst3 = dst.reshape(NT, MSG_CHUNKS, EC)
    dstd = dst.reshape(2 * NT, DEG_CHUNKS, EC)

    hist = _deg_call(dstd)
    dinv = _dinv_call(hist).reshape(NP_PAD, 1)[:N]

    Ws = [W0, W1, W2]
    bs = [b0, b1, b2]
    gs = [gamma0, gamma1, gamma2]
    bts = [beta0, beta1, beta2]

    z = None
    st = None
    for i in range(3):
        if i == 0:
            ylo, yhi = _mm1_call(x, Ws[0], dinv)
        else:
            ylo, yhi = _mm_norm_call(z, st, gs[i - 1].reshape(1, H),
                                     bts[i - 1].reshape(1, H), Ws[i], dinv)
        slo, shi = _msg_call(ylo, yhi, src3, dst3)
        z, st = _combine_call(ylo, yhi, slo, shi, dinv, bs[i].reshape(1, H))

    out = _pool_call(z, st, gs[2].reshape(1, H), bts[2].reshape(1, H),
                     batch.astype(jnp.int32).reshape(N, 1), fc1_W,
                     fc1_b.reshape(1, H // 2), fc2_W, fc2_b.reshape(1, 1))
    return out.reshape(B)
